# baseline (speedup 1.0000x reference)
import functools

import jax
import jax.numpy as jnp
from jax import lax
from jax.experimental import pallas as pl
from jax.experimental.pallas import tpu as pltpu
from jax.experimental.pallas import tpu_sc as plsc

_B, _L, _DIM, _N_EMBED = 8, 1024, 256, 8192
_TBLK = 512
_CCHUNK = 1024
_NCHUNK = _N_EMBED // _CCHUNK


def _argmin_kernel(x_ref, emb_ref, idx_ref, best_ref, e2_ref):
    @pl.when(pl.program_id(0) == 0)
    def _():
        for c in range(_NCHUNK):
            e = emb_ref[:, c * _CCHUNK:(c + 1) * _CCHUNK]
            e2_ref[:, c * _CCHUNK:(c + 1) * _CCHUNK] = jnp.sum(
                e * e, axis=0, keepdims=True)

    x = x_ref[...]
    x2 = x + x
    z2 = jnp.sum(x * x, axis=1, keepdims=True)
    best_d = jnp.full((_TBLK, 1), jnp.inf, jnp.float32)
    best_i = jnp.full((_TBLK, 1), 0.0, jnp.float32)
    ii = lax.broadcasted_iota(
        jnp.int32, (_TBLK, _CCHUNK), 1).astype(jnp.float32)
    for c in range(_NCHUNK):
        e = emb_ref[:, c * _CCHUNK:(c + 1) * _CCHUNK]
        m2 = lax.dot_general(
            x2, e, (((1,), (0,)), ((), ())),
            preferred_element_type=jnp.float32,
            precision=lax.Precision.DEFAULT,
        )
        e2 = e2_ref[:, c * _CCHUNK:(c + 1) * _CCHUNK]
        d = (z2 - m2) + e2
        loc_min = jnp.min(d, axis=1, keepdims=True)
        cand = jnp.where(d == loc_min, ii, jnp.inf)
        loc_arg = jnp.min(cand, axis=1, keepdims=True) + float(c * _CCHUNK)
        upd = loc_min < best_d
        best_i = jnp.where(upd, loc_arg, best_i)
        best_d = jnp.where(upd, loc_min, best_d)
    idx_ref[0, :, :] = best_i.astype(jnp.int32)
    best_ref[0, :, :] = best_d


def _tc_argmin(xf, embed):
    n_blocks = (_B * _L) // _TBLK
    idx, best = pl.pallas_call(
        _argmin_kernel,
        grid=(n_blocks,),
        in_specs=[
            pl.BlockSpec((_TBLK, _DIM), lambda i: (i, 0)),
            pl.BlockSpec((_DIM, _N_EMBED), lambda i: (0, 0)),
        ],
        out_specs=[
            pl.BlockSpec((1, _TBLK, 1), lambda i: (i, 0, 0)),
            pl.BlockSpec((1, _TBLK, 1), lambda i: (i, 0, 0)),
        ],
        out_shape=[
            jax.ShapeDtypeStruct((n_blocks, _TBLK, 1), jnp.int32),
            jax.ShapeDtypeStruct((n_blocks, _TBLK, 1), jnp.float32),
        ],
        scratch_shapes=[pltpu.VMEM((1, _N_EMBED), jnp.float32)],
    )(xf, embed)
    return idx, best


def _make_sc_gather(n_tokens, dim):
    info = plsc.get_sparse_core_info()
    nw = info.num_cores * info.num_subcores
    b_per_w = n_tokens // nw
    mesh = plsc.VectorSubcoreMesh(core_axis_name="c", subcore_axis_name="s")

    @functools.partial(
        pl.kernel,
        mesh=mesh,
        out_type=jax.ShapeDtypeStruct((n_tokens, dim), jnp.float32),
        scratch_types=[
            pltpu.VMEM((b_per_w,), jnp.int32),
            pltpu.VMEM((b_per_w, dim), jnp.float32),
            pltpu.SemaphoreType.DMA,
        ],
    )
    def gather(table_hbm, idx_hbm, out_hbm, idx_v, rows_v, sem):
        wid = lax.axis_index("s") * info.num_cores + lax.axis_index("c")
        base = wid * b_per_w
        pltpu.sync_copy(idx_hbm.at[pl.ds(base, b_per_w)], idx_v)
        pltpu.async_copy(table_hbm.at[idx_v], rows_v, sem).wait()
        pltpu.sync_copy(rows_v, out_hbm.at[pl.ds(base, b_per_w)])

    return gather


_sc_gather = None


def kernel(x, embed):
    global _sc_gather
    if _sc_gather is None:
        _sc_gather = _make_sc_gather(_B * _L, _DIM)
    xf = x.reshape(_B * _L, _DIM)
    idx3, best3 = _tc_argmin(xf, embed)
    idx = idx3.reshape(_B * _L)
    quant = _sc_gather(embed.T, idx)
    quantize = quant.reshape(_B, _L, _DIM)
    diff = jnp.sum(best3) / (_B * _L * _DIM)
    embed_ind = idx.reshape(_B, _L)
    return quantize, diff, embed_ind


# speedup vs baseline: 1.0709x; 1.0709x over previous
import functools

import jax
import jax.numpy as jnp
from jax import lax
from jax.experimental import pallas as pl
from jax.experimental.pallas import tpu as pltpu
from jax.experimental.pallas import tpu_sc as plsc

_B, _L, _DIM, _N_EMBED = 8, 1024, 256, 8192
_TBLK = 1024
_CCHUNK = 1024
_NCHUNK = _N_EMBED // _CCHUNK


def _argmin_kernel(x_ref, emb_ref, idx_ref, best_ref, e2_ref):
    @pl.when(pl.program_id(0) == 0)
    def _():
        for c in range(_NCHUNK):
            e = emb_ref[:, c * _CCHUNK:(c + 1) * _CCHUNK]
            e2_ref[:, c * _CCHUNK:(c + 1) * _CCHUNK] = jnp.sum(
                e * e, axis=0, keepdims=True)

    x = x_ref[...]
    x2 = x + x
    z2 = jnp.sum(x * x, axis=1, keepdims=True)
    best_d = jnp.full((_TBLK, 1), jnp.inf, jnp.float32)
    best_i = jnp.full((_TBLK, 1), 0.0, jnp.float32)
    ii = lax.broadcasted_iota(
        jnp.int32, (_TBLK, _CCHUNK), 1).astype(jnp.float32)
    for c in range(_NCHUNK):
        e = emb_ref[:, c * _CCHUNK:(c + 1) * _CCHUNK]
        m2 = lax.dot_general(
            x2, e, (((1,), (0,)), ((), ())),
            preferred_element_type=jnp.float32,
            precision=lax.Precision.DEFAULT,
        )
        e2 = e2_ref[:, c * _CCHUNK:(c + 1) * _CCHUNK]
        d = (z2 - m2) + e2
        loc_min = jnp.min(d, axis=1, keepdims=True)
        cand = jnp.where(d == loc_min, ii, jnp.inf)
        loc_arg = jnp.min(cand, axis=1, keepdims=True) + float(c * _CCHUNK)
        upd = loc_min < best_d
        best_i = jnp.where(upd, loc_arg, best_i)
        best_d = jnp.where(upd, loc_min, best_d)
    idx_ref[0, :, :] = best_i.astype(jnp.int32)
    best_ref[0, :, :] = best_d


def _tc_argmin(xf, embed):
    n_blocks = (_B * _L) // _TBLK
    idx, best = pl.pallas_call(
        _argmin_kernel,
        grid=(n_blocks,),
        in_specs=[
            pl.BlockSpec((_TBLK, _DIM), lambda i: (i, 0)),
            pl.BlockSpec((_DIM, _N_EMBED), lambda i: (0, 0)),
        ],
        out_specs=[
            pl.BlockSpec((1, _TBLK, 1), lambda i: (i, 0, 0)),
            pl.BlockSpec((1, _TBLK, 1), lambda i: (i, 0, 0)),
        ],
        out_shape=[
            jax.ShapeDtypeStruct((n_blocks, _TBLK, 1), jnp.int32),
            jax.ShapeDtypeStruct((n_blocks, _TBLK, 1), jnp.float32),
        ],
        scratch_shapes=[pltpu.VMEM((1, _N_EMBED), jnp.float32)],
    )(xf, embed)
    return idx, best


def _make_sc_gather(n_tokens, dim):
    info = plsc.get_sparse_core_info()
    nw = info.num_cores * info.num_subcores
    b_per_w = n_tokens // nw
    mesh = plsc.VectorSubcoreMesh(core_axis_name="c", subcore_axis_name="s")

    @functools.partial(
        pl.kernel,
        mesh=mesh,
        out_type=jax.ShapeDtypeStruct((n_tokens, dim), jnp.float32),
        scratch_types=[
            pltpu.VMEM((b_per_w,), jnp.int32),
            pltpu.VMEM((b_per_w, dim), jnp.float32),
            pltpu.SemaphoreType.DMA,
        ],
    )
    def gather(table_hbm, idx_hbm, out_hbm, idx_v, rows_v, sem):
        wid = lax.axis_index("s") * info.num_cores + lax.axis_index("c")
        base = wid * b_per_w
        pltpu.sync_copy(idx_hbm.at[pl.ds(base, b_per_w)], idx_v)
        pltpu.async_copy(table_hbm.at[idx_v], rows_v, sem).wait()
        pltpu.sync_copy(rows_v, out_hbm.at[pl.ds(base, b_per_w)])

    return gather


_sc_gather = None


def kernel(x, embed):
    global _sc_gather
    if _sc_gather is None:
        _sc_gather = _make_sc_gather(_B * _L, _DIM)
    xf = x.reshape(_B * _L, _DIM)
    idx3, best3 = _tc_argmin(xf, embed)
    idx = idx3.reshape(_B * _L)
    quant = _sc_gather(embed.T, idx)
    quantize = quant.reshape(_B, _L, _DIM)
    diff = jnp.sum(best3) / (_B * _L * _DIM)
    embed_ind = idx.reshape(_B, _L)
    return quantize, diff, embed_ind
